# Initial kernel scaffold; baseline (speedup 1.0000x reference)
import functools

import jax
import jax.numpy as jnp
from jax import lax
from jax.experimental import pallas as pl
from jax.experimental.pallas import tpu as pltpu
from jax.experimental.pallas import tpu_sc as plsc

NN = 10000
EE = 320000
DD = 128
NP = 10240
ET = EE + NN
NC = 2
NS = 16
NW = NC * NS
CH = 81
EP = CH * 128
ETP = NW * EP
RPT = NP // NS


def _proj_body(x_ref, w_ref, a_ref, h_ref, ab_ref):
    hb = jnp.dot(x_ref[...], w_ref[...], preferred_element_type=jnp.float32)
    h_ref[...] = hb
    ab_ref[...] = jnp.dot(hb, a_ref[...], preferred_element_type=jnp.float32)


def _proj(xp, W, A2p):
    blk = 512
    grid = NP // blk
    return pl.pallas_call(
        _proj_body,
        grid=(grid,),
        in_specs=[
            pl.BlockSpec((blk, DD), lambda i: (i, 0)),
            pl.BlockSpec((DD, DD), lambda i: (0, 0)),
            pl.BlockSpec((DD, DD), lambda i: (0, 0)),
        ],
        out_specs=[
            pl.BlockSpec((blk, DD), lambda i: (i, 0)),
            pl.BlockSpec((blk, DD), lambda i: (i, 0)),
        ],
        out_shape=[
            jax.ShapeDtypeStruct((NP, DD), jnp.float32),
            jax.ShapeDtypeStruct((NP, DD), jnp.float32),
        ],
    )(xp, W, A2p)


def _edge_body(src_hbm, dst_hbm, h_hbm, asv_hbm, adv_hbm, nump_hbm, denp_hbm,
               sidx2, didx2, wf, asv_v, adv_v, rows_v, zvec, num_sh, den_sh,
               sem):
    c = lax.axis_index("c")
    s = lax.axis_index("s")
    wid = c * NS + s

    pltpu.sync_copy(src_hbm.at[wid], sidx2)
    pltpu.sync_copy(dst_hbm.at[wid], didx2)
    pltpu.sync_copy(asv_hbm, asv_v)
    pltpu.sync_copy(adv_hbm, adv_v)

    zero16 = jnp.zeros((16,), jnp.float32)

    def _zrow(r, carry):
        for l in range(8):
            rows_v[r, pl.ds(l * 16, 16)] = zero16
        return carry

    lax.fori_loop(0, 128, _zrow, 0)

    def _zv(i, carry):
        zvec[pl.ds(i * 16, 16)] = zero16
        return carry

    lax.fori_loop(0, RPT // 16, _zv, 0)

    for k in range(RPT // 128):
        pltpu.sync_copy(rows_v, num_sh.at[pl.ds(s * RPT + k * 128, 128)])
    pltpu.sync_copy(zvec, den_sh.at[pl.ds(s * RPT, RPT)])
    plsc.subcore_barrier()

    ebase = wid * EP

    def _chunk(j, carry):
        for i in range(8):
            off = j * 128 + i * 16
            s16 = sidx2[j, pl.ds(i * 16, 16)]
            d16 = didx2[j, pl.ds(i * 16, 16)]
            a = plsc.load_gather(asv_v, [s16]) + plsc.load_gather(adv_v, [d16])
            a = jnp.where(a > 0, a, a * jnp.float32(0.2))
            w = jnp.exp(a)
            eid = ebase + off + lax.broadcasted_iota(jnp.int32, (16,), 0)
            w = jnp.where(eid < ET, w, jnp.float32(0.0))
            wf[pl.ds(off, 16)] = w

        pltpu.sync_copy(wf.at[pl.ds(j * 128, 128)], den_sh.at[didx2.at[j]],
                        add=True)

        pltpu.async_copy(h_hbm.at[sidx2.at[j]], rows_v, sem).wait()

        def _srow(r, carry2):
            wspl = plsc.load_gather(
                wf, [jnp.full((16,), j * 128 + r, jnp.int32)])
            for l in range(8):
                rows_v[r, pl.ds(l * 16, 16)] = (
                    rows_v[r, pl.ds(l * 16, 16)] * wspl)
            return carry2

        lax.fori_loop(0, 128, _srow, 0)
        pltpu.sync_copy(rows_v, num_sh.at[didx2.at[j]], add=True)
        return carry

    lax.fori_loop(0, CH, _chunk, 0)
    plsc.subcore_barrier()

    pltpu.sync_copy(num_sh.at[pl.ds(s * RPT, RPT)], nump_hbm.at[c, s])
    pltpu.sync_copy(den_sh.at[pl.ds(s * RPT, RPT)], denp_hbm.at[c, s])


def _edge(srcp, dstp, h, asv, adv):
    mesh = plsc.VectorSubcoreMesh(core_axis_name="c", subcore_axis_name="s",
                                  num_cores=NC, num_subcores=NS)
    f = pl.kernel(
        _edge_body,
        out_type=[
            jax.ShapeDtypeStruct((NC, NS, RPT, DD), jnp.float32),
            jax.ShapeDtypeStruct((NC, NS, RPT), jnp.float32),
        ],
        mesh=mesh,
        scratch_types=[
            pltpu.VMEM((CH, 128), jnp.int32),
            pltpu.VMEM((CH, 128), jnp.int32),
            pltpu.VMEM((EP,), jnp.float32),
            pltpu.VMEM((NP,), jnp.float32),
            pltpu.VMEM((NP,), jnp.float32),
            pltpu.VMEM((128, DD), jnp.float32),
            pltpu.VMEM((RPT,), jnp.float32),
            pltpu.VMEM_SHARED((NP, DD), jnp.float32),
            pltpu.VMEM_SHARED((NP,), jnp.float32),
            pltpu.SemaphoreType.DMA,
        ],
    )
    return f(srcp, dstp, h, asv, adv)


def _comb_body(n0_ref, n1_ref, d0_ref, d1_ref, b_ref, o_ref):
    d = d0_ref[...] + d1_ref[...]
    o_ref[...] = (n0_ref[...] + n1_ref[...]) / (d + jnp.float32(1e-16)) \
        + b_ref[...]


def _combine(num0, num1, den0, den1, biasr):
    blk = 1000
    grid = NN // blk
    return pl.pallas_call(
        _comb_body,
        grid=(grid,),
        in_specs=[
            pl.BlockSpec((blk, DD), lambda i: (i, 0)),
            pl.BlockSpec((blk, DD), lambda i: (i, 0)),
            pl.BlockSpec((blk, 1), lambda i: (i, 0)),
            pl.BlockSpec((blk, 1), lambda i: (i, 0)),
            pl.BlockSpec((1, DD), lambda i: (0, 0)),
        ],
        out_specs=pl.BlockSpec((blk, DD), lambda i: (i, 0)),
        out_shape=jax.ShapeDtypeStruct((NN, DD), jnp.float32),
    )(num0, num1, den0, den1, biasr)


def kernel(x, edge_index, W, att_src, att_dst, bias):
    xp = jnp.pad(x, ((0, NP - NN), (0, 0)))
    A2p = jnp.zeros((DD, DD), jnp.float32)
    A2p = A2p.at[:, 0].set(att_src.reshape(DD))
    A2p = A2p.at[:, 1].set(att_dst.reshape(DD))

    h, ab = _proj(xp, W, A2p)
    asv = ab[:, 0]
    adv = ab[:, 1]

    loop = jnp.arange(NN, dtype=jnp.int32)
    pad = jnp.zeros((ETP - ET,), jnp.int32)
    srcp = jnp.concatenate([edge_index[0], loop, pad]).reshape(NW, CH, 128)
    dstp = jnp.concatenate([edge_index[1], loop, pad]).reshape(NW, CH, 128)

    nump, denp = _edge(srcp, dstp, h, asv, adv)
    nump = nump.reshape(NC, NP, DD)
    denp = denp.reshape(NC, NP, 1)

    out = _combine(nump[0], nump[1], denp[0], denp[1], bias.reshape(1, DD))
    return out


# speedup vs baseline: 21.9345x; 21.9345x over previous
import functools

import jax
import jax.numpy as jnp
from jax import lax
from jax.experimental import pallas as pl
from jax.experimental.pallas import tpu as pltpu
from jax.experimental.pallas import tpu_sc as plsc

NN = 10000
EE = 320000
DD = 128
NP = 10240
ET = EE + NN
NC = 2
NS = 16
NW = NC * NS
CH = 81
EP = CH * 128
ETP = NW * EP
RPT = NP // NS


def _proj_body(x_ref, w_ref, a_ref, h_ref, ab_ref):
    hb = jnp.dot(x_ref[...], w_ref[...], preferred_element_type=jnp.float32)
    h_ref[...] = hb
    ab_ref[...] = jnp.dot(hb, a_ref[...], preferred_element_type=jnp.float32)


def _proj(xp, W, A2p):
    blk = 512
    grid = NP // blk
    return pl.pallas_call(
        _proj_body,
        grid=(grid,),
        in_specs=[
            pl.BlockSpec((blk, DD), lambda i: (i, 0)),
            pl.BlockSpec((DD, DD), lambda i: (0, 0)),
            pl.BlockSpec((DD, DD), lambda i: (0, 0)),
        ],
        out_specs=[
            pl.BlockSpec((blk, DD), lambda i: (i, 0)),
            pl.BlockSpec((blk, DD), lambda i: (i, 0)),
        ],
        out_shape=[
            jax.ShapeDtypeStruct((NP, DD), jnp.float32),
            jax.ShapeDtypeStruct((NP, DD), jnp.float32),
        ],
    )(xp, W, A2p)


def _edge_body(src_hbm, dst_hbm, h_hbm, asv_hbm, adv_hbm, nump_hbm, denp_hbm,
               sidxc, didxc, wfc, asv_v, adv_v, rows_v, zvec, num_sh, den_sh,
               sem):
    c = lax.axis_index("c")
    s = lax.axis_index("s")
    wid = c * NS + s

    pltpu.sync_copy(asv_hbm, asv_v)
    pltpu.sync_copy(adv_hbm, adv_v)

    zero16 = jnp.zeros((16,), jnp.float32)

    def _zrow(r, carry):
        for l in range(8):
            rows_v[r, pl.ds(l * 16, 16)] = zero16
        return carry

    lax.fori_loop(0, 128, _zrow, 0)

    def _zv(i, carry):
        zvec[pl.ds(i * 16, 16)] = zero16
        return carry

    lax.fori_loop(0, RPT // 16, _zv, 0)

    for k in range(RPT // 128):
        pltpu.sync_copy(rows_v, num_sh.at[pl.ds(s * RPT + k * 128, 128)])
    pltpu.sync_copy(zvec, den_sh.at[pl.ds(s * RPT, RPT)])
    plsc.subcore_barrier()

    ebase = wid * EP

    def _chunk(j, carry):
        pltpu.sync_copy(src_hbm.at[wid].at[pl.ds(j, 1)], sidxc)
        pltpu.sync_copy(dst_hbm.at[wid].at[pl.ds(j, 1)], didxc)

        for i in range(8):
            s16 = sidxc[0, pl.ds(i * 16, 16)]
            d16 = didxc[0, pl.ds(i * 16, 16)]
            a = plsc.load_gather(asv_v, [s16]) + plsc.load_gather(adv_v, [d16])
            a = jnp.where(a > 0, a, a * jnp.float32(0.2))
            w = jnp.exp(a)
            eid = (ebase + j * 128 + i * 16
                   + lax.broadcasted_iota(jnp.int32, (16,), 0))
            w = jnp.where(eid < ET, w, jnp.float32(0.0))
            wfc[pl.ds(i * 16, 16)] = w

        pltpu.sync_copy(wfc, den_sh.at[didxc.at[0]], add=True)

        pltpu.async_copy(h_hbm.at[sidxc.at[0]], rows_v, sem).wait()

        def _srow(r, carry2):
            wspl = plsc.load_gather(wfc, [jnp.full((16,), r, jnp.int32)])
            for l in range(8):
                rows_v[r, pl.ds(l * 16, 16)] = (
                    rows_v[r, pl.ds(l * 16, 16)] * wspl)
            return carry2

        lax.fori_loop(0, 128, _srow, 0)
        pltpu.sync_copy(rows_v, num_sh.at[didxc.at[0]], add=True)
        return carry

    lax.fori_loop(0, CH, _chunk, 0)
    plsc.subcore_barrier()

    pltpu.sync_copy(num_sh.at[pl.ds(s * RPT, RPT)], nump_hbm.at[c, s])
    pltpu.sync_copy(den_sh.at[pl.ds(s * RPT, RPT)], denp_hbm.at[c, s])


def _edge(srcp, dstp, h, asv, adv):
    mesh = plsc.VectorSubcoreMesh(core_axis_name="c", subcore_axis_name="s",
                                  num_cores=NC, num_subcores=NS)
    f = pl.kernel(
        _edge_body,
        out_type=[
            jax.ShapeDtypeStruct((NC, NS, RPT, DD), jnp.float32),
            jax.ShapeDtypeStruct((NC, NS, RPT), jnp.float32),
        ],
        mesh=mesh,
        compiler_params=pltpu.CompilerParams(needs_layout_passes=False),
        scratch_types=[
            pltpu.VMEM((1, 128), jnp.int32),
            pltpu.VMEM((1, 128), jnp.int32),
            pltpu.VMEM((128,), jnp.float32),
            pltpu.VMEM((NP,), jnp.float32),
            pltpu.VMEM((NP,), jnp.float32),
            pltpu.VMEM((128, DD), jnp.float32),
            pltpu.VMEM((RPT,), jnp.float32),
            pltpu.VMEM_SHARED((NP, DD), jnp.float32),
            pltpu.VMEM_SHARED((NP,), jnp.float32),
            pltpu.SemaphoreType.DMA,
        ],
    )
    return f(srcp, dstp, h, asv, adv)


def _comb_body(n0_ref, n1_ref, d0_ref, d1_ref, b_ref, o_ref):
    d = d0_ref[...] + d1_ref[...]
    o_ref[...] = (n0_ref[...] + n1_ref[...]) / (d + jnp.float32(1e-16)) \
        + b_ref[...]


def _combine(num0, num1, den0, den1, biasr):
    blk = 1000
    grid = NN // blk
    return pl.pallas_call(
        _comb_body,
        grid=(grid,),
        in_specs=[
            pl.BlockSpec((blk, DD), lambda i: (i, 0)),
            pl.BlockSpec((blk, DD), lambda i: (i, 0)),
            pl.BlockSpec((blk, 1), lambda i: (i, 0)),
            pl.BlockSpec((blk, 1), lambda i: (i, 0)),
            pl.BlockSpec((1, DD), lambda i: (0, 0)),
        ],
        out_specs=pl.BlockSpec((blk, DD), lambda i: (i, 0)),
        out_shape=jax.ShapeDtypeStruct((NN, DD), jnp.float32),
    )(num0, num1, den0, den1, biasr)


def kernel(x, edge_index, W, att_src, att_dst, bias):
    xp = jnp.pad(x, ((0, NP - NN), (0, 0)))
    A2p = jnp.zeros((DD, DD), jnp.float32)
    A2p = A2p.at[:, 0].set(att_src.reshape(DD))
    A2p = A2p.at[:, 1].set(att_dst.reshape(DD))

    h, ab = _proj(xp, W, A2p)
    asv = ab[:, 0]
    adv = ab[:, 1]

    loop = jnp.arange(NN, dtype=jnp.int32)
    pad = jnp.zeros((ETP - ET,), jnp.int32)
    srcp = jnp.concatenate([edge_index[0], loop, pad]).reshape(NW, CH, 128)
    dstp = jnp.concatenate([edge_index[1], loop, pad]).reshape(NW, CH, 128)

    nump, denp = _edge(srcp, dstp, h, asv, adv)
    nump = nump.reshape(NC, NP, DD)
    denp = denp.reshape(NC, NP, 1)

    out = _combine(nump[0], nump[1], denp[0], denp[1], bias.reshape(1, DD))
    return out


# speedup vs baseline: 30.3626x; 1.3842x over previous
import functools

import jax
import jax.numpy as jnp
from jax import lax
from jax.experimental import pallas as pl
from jax.experimental.pallas import tpu as pltpu
from jax.experimental.pallas import tpu_sc as plsc

NN = 10000
EE = 320000
DD = 128
NP = 10240
ET = EE + NN
NC = 2
NS = 16
NW = NC * NS
CK = 64
CH = 162
EP = CH * CK
ETP = NW * EP
RPT = NP // NS
SUP = 9
NSUP = CH // SUP
LA = 2


def _proj_body(x_ref, w_ref, a_ref, h_ref, ab_ref):
    hb = jnp.dot(x_ref[...], w_ref[...], preferred_element_type=jnp.float32)
    h_ref[...] = hb
    ab_ref[...] = jnp.dot(hb, a_ref[...], preferred_element_type=jnp.float32)


def _proj(xp, W, A2p):
    blk = 512
    grid = NP // blk
    return pl.pallas_call(
        _proj_body,
        grid=(grid,),
        in_specs=[
            pl.BlockSpec((blk, DD), lambda i: (i, 0)),
            pl.BlockSpec((DD, DD), lambda i: (0, 0)),
            pl.BlockSpec((DD, DD), lambda i: (0, 0)),
        ],
        out_specs=[
            pl.BlockSpec((blk, DD), lambda i: (i, 0)),
            pl.BlockSpec((blk, DD), lambda i: (i, 0)),
        ],
        out_shape=[
            jax.ShapeDtypeStruct((NP, DD), jnp.float32),
            jax.ShapeDtypeStruct((NP, DD), jnp.float32),
        ],
    )(xp, W, A2p)


def _edge_body(src_hbm, dst_hbm, h_hbm, asv_hbm, adv_hbm, nump_hbm, denp_hbm,
               sidxsup, didxsup,
               sidx0, sidx1, sidx2, didx0, didx1, didx2,
               wf0, wf1, wf2, rows0, rows1, rows2,
               asv_v, adv_v, zvec, num_sh, den_sh,
               gsem0, gsem1, gsem2, nsem0, nsem1, nsem2,
               dsem0, dsem1, dsem2):
    c = lax.axis_index("c")
    s = lax.axis_index("s")
    wid = c * NS + s
    ebase = wid * EP

    sidx = (sidx0, sidx1, sidx2)
    didx = (didx0, didx1, didx2)
    wf = (wf0, wf1, wf2)
    rows = (rows0, rows1, rows2)
    gsem = (gsem0, gsem1, gsem2)
    nsem = (nsem0, nsem1, nsem2)
    dsem = (dsem0, dsem1, dsem2)

    pltpu.sync_copy(asv_hbm, asv_v)
    pltpu.sync_copy(adv_hbm, adv_v)

    zero16 = jnp.zeros((16,), jnp.float32)

    def _zrow(r, carry):
        for l in range(8):
            rows0[r, pl.ds(l * 16, 16)] = zero16
        return carry

    lax.fori_loop(0, CK, _zrow, 0)

    def _zv(i, carry):
        zvec[pl.ds(i * 16, 16)] = zero16
        return carry

    lax.fori_loop(0, RPT // 16, _zv, 0)

    for m in range(RPT // CK):
        pltpu.sync_copy(rows0, num_sh.at[pl.ds(s * RPT + m * CK, CK)])
    pltpu.sync_copy(zvec, den_sh.at[pl.ds(s * RPT, RPT)])
    plsc.subcore_barrier()

    def _stage_super(su):
        pltpu.sync_copy(src_hbm.at[wid].at[pl.ds(su, 1)], sidxsup)
        pltpu.sync_copy(dst_hbm.at[wid].at[pl.ds(su, 1)], didxsup)

    def _front(f, k):
        p = f % SUP
        ws = []
        for q in range(CK // 16):
            s16 = sidxsup[0, pl.ds(p * CK + q * 16, 16)]
            d16 = didxsup[0, pl.ds(p * CK + q * 16, 16)]
            sidx[k][0, pl.ds(q * 16, 16)] = s16
            didx[k][0, pl.ds(q * 16, 16)] = d16
            a = plsc.load_gather(asv_v, [s16]) + plsc.load_gather(adv_v, [d16])
            ws.append((a, f * CK + q * 16))
        pltpu.async_copy(h_hbm.at[sidx[k].at[0]], rows[k], gsem[k])
        for q in range(CK // 16):
            a, off = ws[q]
            a = jnp.where(a > 0, a, a * jnp.float32(0.2))
            w = jnp.exp(a)
            eid = ebase + off + lax.broadcasted_iota(jnp.int32, (16,), 0)
            w = jnp.where(eid < ET, w, jnp.float32(0.0))
            wf[k][pl.ds(q * 16, 16)] = w

    def _wait_scatters(k):
        pass

    def _back(k):
        pltpu.make_async_copy(h_hbm.at[sidx[k].at[0]], rows[k],
                              gsem[k]).wait()

        def _srow(r, carry2):
            wspl = plsc.load_gather(wf[k], [jnp.full((16,), r, jnp.int32)])
            for l in range(8):
                rows[k][r, pl.ds(l * 16, 16)] = (
                    rows[k][r, pl.ds(l * 16, 16)] * wspl)
            return carry2

        lax.fori_loop(0, CK, _srow, 0)
        pltpu.sync_copy(rows[k], num_sh.at[didx[k].at[0]], add=True)
        pltpu.sync_copy(wf[k], den_sh.at[didx[k].at[0]], add=True)

    _stage_super(0)
    _front(0, 0)
    _front(1, 1)
    _front(2, 2)
    _back(0)

    def _iter(js, carry):
        for k in range(3):
            f = 3 * js + k
            if k == 0:
                pl.when(js % 3 == 0)(lambda: _stage_super(js // 3))
            _wait_scatters(k)
            _front(f, k)
            _back((k + 1) % 3)
        return carry

    lax.fori_loop(1, CH // 3, _iter, 0)

    _back(1)
    _back(2)
    for k in range(3):
        _wait_scatters(k)
    plsc.subcore_barrier()

    pltpu.sync_copy(num_sh.at[pl.ds(s * RPT, RPT)], nump_hbm.at[c, s])
    pltpu.sync_copy(den_sh.at[pl.ds(s * RPT, RPT)], denp_hbm.at[c, s])


def _edge(srcp, dstp, h, asv, adv):
    mesh = plsc.VectorSubcoreMesh(core_axis_name="c", subcore_axis_name="s",
                                  num_cores=NC, num_subcores=NS)
    f = pl.kernel(
        _edge_body,
        out_type=[
            jax.ShapeDtypeStruct((NC, NS, RPT, DD), jnp.float32),
            jax.ShapeDtypeStruct((NC, NS, RPT), jnp.float32),
        ],
        mesh=mesh,
        compiler_params=pltpu.CompilerParams(needs_layout_passes=False),
        scratch_types=(
            [pltpu.VMEM((1, SUP * CK), jnp.int32)] * 2
            + [pltpu.VMEM((1, CK), jnp.int32)] * 6
            + [pltpu.VMEM((CK,), jnp.float32)] * 3
            + [pltpu.VMEM((CK, DD), jnp.float32)] * 3
            + [pltpu.VMEM((NN,), jnp.float32)] * 2
            + [pltpu.VMEM((RPT,), jnp.float32)]
            + [pltpu.VMEM_SHARED((NP, DD), jnp.float32)]
            + [pltpu.VMEM_SHARED((NP,), jnp.float32)]
            + [pltpu.SemaphoreType.DMA] * 9
        ),
    )
    return f(srcp, dstp, h, asv, adv)


def _comb_body(n0_ref, n1_ref, d0_ref, d1_ref, b_ref, o_ref):
    d = d0_ref[...] + d1_ref[...]
    o_ref[...] = (n0_ref[...] + n1_ref[...]) / (d + jnp.float32(1e-16)) \
        + b_ref[...]


def _combine(num0, num1, den0, den1, biasr):
    blk = 1000
    grid = NN // blk
    return pl.pallas_call(
        _comb_body,
        grid=(grid,),
        in_specs=[
            pl.BlockSpec((blk, DD), lambda i: (i, 0)),
            pl.BlockSpec((blk, DD), lambda i: (i, 0)),
            pl.BlockSpec((blk, 1), lambda i: (i, 0)),
            pl.BlockSpec((blk, 1), lambda i: (i, 0)),
            pl.BlockSpec((1, DD), lambda i: (0, 0)),
        ],
        out_specs=pl.BlockSpec((blk, DD), lambda i: (i, 0)),
        out_shape=jax.ShapeDtypeStruct((NN, DD), jnp.float32),
    )(num0, num1, den0, den1, biasr)


def kernel(x, edge_index, W, att_src, att_dst, bias):
    xp = jnp.pad(x, ((0, NP - NN), (0, 0)))
    A2p = jnp.zeros((DD, DD), jnp.float32)
    A2p = A2p.at[:, 0].set(att_src.reshape(DD))
    A2p = A2p.at[:, 1].set(att_dst.reshape(DD))

    h, ab = _proj(xp, W, A2p)
    asv = ab[:NN, 0]
    adv = ab[:NN, 1]

    loop = jnp.arange(NN, dtype=jnp.int32)
    pad = jnp.zeros((ETP - ET,), jnp.int32)
    srcp = jnp.concatenate([edge_index[0], loop, pad]).reshape(
        NW, NSUP, SUP * CK)
    dstp = jnp.concatenate([edge_index[1], loop, pad]).reshape(
        NW, NSUP, SUP * CK)

    nump, denp = _edge(srcp, dstp, h, asv, adv)
    nump = nump.reshape(NC, NP, DD)
    denp = denp.reshape(NC, NP, 1)

    out = _combine(nump[0], nump[1], denp[0], denp[1], bias.reshape(1, DD))
    return out


# speedup vs baseline: 31.3567x; 1.0327x over previous
import functools

import jax
import jax.numpy as jnp
from jax import lax
from jax.experimental import pallas as pl
from jax.experimental.pallas import tpu as pltpu
from jax.experimental.pallas import tpu_sc as plsc

NN = 10000
EE = 320000
DD = 128
NP = 10240
ET = EE + NN
NC = 2
NS = 16
NW = NC * NS
CK = 64
CH = 162
EP = CH * CK
ETP = NW * EP
RPT = NP // NS
SUP = 9
NSUP = CH // SUP
LA = 2


def _proj_body(x_ref, w_ref, a_ref, h_ref, ab_ref):
    hb = jnp.dot(x_ref[...], w_ref[...], preferred_element_type=jnp.float32)
    h_ref[...] = hb
    ab_ref[...] = jnp.dot(hb, a_ref[...], preferred_element_type=jnp.float32)


def _proj(xp, W, A2p):
    blk = 512
    grid = NP // blk
    return pl.pallas_call(
        _proj_body,
        grid=(grid,),
        in_specs=[
            pl.BlockSpec((blk, DD), lambda i: (i, 0)),
            pl.BlockSpec((DD, DD), lambda i: (0, 0)),
            pl.BlockSpec((DD, DD), lambda i: (0, 0)),
        ],
        out_specs=[
            pl.BlockSpec((blk, DD), lambda i: (i, 0)),
            pl.BlockSpec((blk, DD), lambda i: (i, 0)),
        ],
        out_shape=[
            jax.ShapeDtypeStruct((NP, DD), jnp.float32),
            jax.ShapeDtypeStruct((NP, DD), jnp.float32),
        ],
    )(xp, W, A2p)


def _edge_body(src_hbm, dst_hbm, h_hbm, asv_hbm, adv_hbm, nump_hbm, denp_hbm,
               sidxsup, didxsup,
               sidx0, sidx1, sidx2, didx0, didx1, didx2,
               wf0, wf1, wf2, rows0, rows1, rows2,
               asv_v, adv_v, zvec, num_sh, den_sh,
               gsem0, gsem1, gsem2, nsem0, nsem1, nsem2,
               dsem0, dsem1, dsem2):
    c = lax.axis_index("c")
    s = lax.axis_index("s")
    wid = c * NS + s
    ebase = wid * EP

    sidx = (sidx0, sidx1, sidx2)
    didx = (didx0, didx1, didx2)
    wf = (wf0, wf1, wf2)
    rows = (rows0, rows1, rows2)
    gsem = (gsem0, gsem1, gsem2)
    nsem = (nsem0, nsem1, nsem2)
    dsem = (dsem0, dsem1, dsem2)

    pltpu.sync_copy(asv_hbm, asv_v)
    pltpu.sync_copy(adv_hbm, adv_v)

    zero16 = jnp.zeros((16,), jnp.float32)

    def _zrow(r, carry):
        for l in range(8):
            rows0[r, pl.ds(l * 16, 16)] = zero16
        return carry

    lax.fori_loop(0, CK, _zrow, 0)

    def _zv(i, carry):
        zvec[pl.ds(i * 16, 16)] = zero16
        return carry

    lax.fori_loop(0, RPT // 16, _zv, 0)

    for m in range(RPT // CK):
        pltpu.sync_copy(rows0, num_sh.at[pl.ds(s * RPT + m * CK, CK)])
    pltpu.sync_copy(zvec, den_sh.at[pl.ds(s * RPT, RPT)])
    plsc.subcore_barrier()

    def _stage_super(su):
        pltpu.sync_copy(src_hbm.at[wid].at[pl.ds(su, 1)], sidxsup)
        pltpu.sync_copy(dst_hbm.at[wid].at[pl.ds(su, 1)], didxsup)

    def _front(f, k):
        p = f % SUP
        ws = []
        for q in range(CK // 16):
            s16 = sidxsup[0, pl.ds(p * CK + q * 16, 16)]
            d16 = didxsup[0, pl.ds(p * CK + q * 16, 16)]
            sidx[k][0, pl.ds(q * 16, 16)] = s16
            didx[k][0, pl.ds(q * 16, 16)] = d16
            a = plsc.load_gather(asv_v, [s16]) + plsc.load_gather(adv_v, [d16])
            ws.append((a, f * CK + q * 16))
        pltpu.async_copy(h_hbm.at[sidx[k].at[0]], rows[k], gsem[k])
        for q in range(CK // 16):
            a, off = ws[q]
            a = jnp.where(a > 0, a, a * jnp.float32(0.2))
            w = jnp.exp(a)
            eid = ebase + off + lax.broadcasted_iota(jnp.int32, (16,), 0)
            w = jnp.where(eid < ET, w, jnp.float32(0.0))
            wf[k][pl.ds(q * 16, 16)] = w

    def _wait_scatters(k):
        pltpu.make_async_copy(rows[k], num_sh.at[didx[k].at[0]],
                              nsem[k]).wait()
        pltpu.make_async_copy(wf[k], den_sh.at[didx[k].at[0]],
                              dsem[k]).wait()

    def _back(k):
        pltpu.make_async_copy(h_hbm.at[sidx[k].at[0]], rows[k],
                              gsem[k]).wait()

        def _srow(r, carry2):
            wspl = plsc.load_gather(wf[k], [jnp.full((16,), r, jnp.int32)])
            for l in range(8):
                rows[k][r, pl.ds(l * 16, 16)] = (
                    rows[k][r, pl.ds(l * 16, 16)] * wspl)
            return carry2

        lax.fori_loop(0, CK, _srow, 0)
        pltpu.async_copy(rows[k], num_sh.at[didx[k].at[0]], nsem[k], add=True)
        pltpu.async_copy(wf[k], den_sh.at[didx[k].at[0]], dsem[k], add=True)

    _stage_super(0)
    _front(0, 0)
    _front(1, 1)
    _front(2, 2)
    _back(0)

    def _iter(js, carry):
        for k in range(3):
            f = 3 * js + k
            if k == 0:
                pl.when(js % 3 == 0)(lambda: _stage_super(js // 3))
            _wait_scatters(k)
            _front(f, k)
            _back((k + 1) % 3)
        return carry

    lax.fori_loop(1, CH // 3, _iter, 0)

    _back(1)
    _back(2)
    for k in range(3):
        _wait_scatters(k)
    plsc.subcore_barrier()

    pltpu.sync_copy(num_sh.at[pl.ds(s * RPT, RPT)], nump_hbm.at[c, s])
    pltpu.sync_copy(den_sh.at[pl.ds(s * RPT, RPT)], denp_hbm.at[c, s])


def _edge(srcp, dstp, h, asv, adv):
    mesh = plsc.VectorSubcoreMesh(core_axis_name="c", subcore_axis_name="s",
                                  num_cores=NC, num_subcores=NS)
    f = pl.kernel(
        _edge_body,
        out_type=[
            jax.ShapeDtypeStruct((NC, NS, RPT, DD), jnp.float32),
            jax.ShapeDtypeStruct((NC, NS, RPT), jnp.float32),
        ],
        mesh=mesh,
        compiler_params=pltpu.CompilerParams(needs_layout_passes=False),
        scratch_types=(
            [pltpu.VMEM((1, SUP * CK), jnp.int32)] * 2
            + [pltpu.VMEM((1, CK), jnp.int32)] * 6
            + [pltpu.VMEM((CK,), jnp.float32)] * 3
            + [pltpu.VMEM((CK, DD), jnp.float32)] * 3
            + [pltpu.VMEM((NN,), jnp.float32)] * 2
            + [pltpu.VMEM((RPT,), jnp.float32)]
            + [pltpu.VMEM_SHARED((NP, DD), jnp.float32)]
            + [pltpu.VMEM_SHARED((NP,), jnp.float32)]
            + [pltpu.SemaphoreType.DMA] * 9
        ),
    )
    return f(srcp, dstp, h, asv, adv)


def _comb_body(n0_ref, n1_ref, d0_ref, d1_ref, b_ref, o_ref):
    d = d0_ref[...] + d1_ref[...]
    o_ref[...] = (n0_ref[...] + n1_ref[...]) / (d + jnp.float32(1e-16)) \
        + b_ref[...]


def _combine(num0, num1, den0, den1, biasr):
    blk = 1000
    grid = NN // blk
    return pl.pallas_call(
        _comb_body,
        grid=(grid,),
        in_specs=[
            pl.BlockSpec((blk, DD), lambda i: (i, 0)),
            pl.BlockSpec((blk, DD), lambda i: (i, 0)),
            pl.BlockSpec((blk, 1), lambda i: (i, 0)),
            pl.BlockSpec((blk, 1), lambda i: (i, 0)),
            pl.BlockSpec((1, DD), lambda i: (0, 0)),
        ],
        out_specs=pl.BlockSpec((blk, DD), lambda i: (i, 0)),
        out_shape=jax.ShapeDtypeStruct((NN, DD), jnp.float32),
    )(num0, num1, den0, den1, biasr)


def kernel(x, edge_index, W, att_src, att_dst, bias):
    xp = jnp.pad(x, ((0, NP - NN), (0, 0)))
    A2p = jnp.zeros((DD, DD), jnp.float32)
    A2p = A2p.at[:, 0].set(att_src.reshape(DD))
    A2p = A2p.at[:, 1].set(att_dst.reshape(DD))

    h, ab = _proj(xp, W, A2p)
    asv = ab[:NN, 0]
    adv = ab[:NN, 1]

    loop = jnp.arange(NN, dtype=jnp.int32)
    pad = jnp.zeros((ETP - ET,), jnp.int32)
    srcp = jnp.concatenate([edge_index[0], loop, pad]).reshape(
        NW, NSUP, SUP * CK)
    dstp = jnp.concatenate([edge_index[1], loop, pad]).reshape(
        NW, NSUP, SUP * CK)

    nump, denp = _edge(srcp, dstp, h, asv, adv)
    nump = nump.reshape(NC, NP, DD)
    denp = denp.reshape(NC, NP, 1)

    out = _combine(nump[0], nump[1], denp[0], denp[1], bias.reshape(1, DD))
    return out
